# baseline (speedup 1.0000x reference)
import jax
import jax.numpy as jnp
from jax import lax
from jax.experimental import pallas as pl
from jax.experimental.pallas import tpu as pltpu
from jax.experimental.pallas import tpu_sc as plsc

_W = 4
_NUM_TILES = 32


def _copy_body(x_hbm, out_hbm):
    wid = lax.axis_index("s") * 2 + lax.axis_index("c")
    rows = out_hbm.shape[0] // _NUM_TILES
    base = wid * rows
    pltpu.sync_copy(
        x_hbm.at[pl.ds(base, rows), pl.ds(0, 1)],
        out_hbm.at[pl.ds(base, rows)],
    )


def kernel(x):
    b, s, d = x.shape
    h = s // _W
    xv = x.reshape(b * h, _W, d)
    mesh = plsc.VectorSubcoreMesh(core_axis_name="c", subcore_axis_name="s")
    out = pl.kernel(
        _copy_body,
        out_type=jax.ShapeDtypeStruct((b * h, 1, d), x.dtype),
        mesh=mesh,
    )(xv)
    return out.reshape(b, h, d)


# speedup vs baseline: 5.8648x; 5.8648x over previous
import jax
import jax.numpy as jnp
from jax import lax
from jax.experimental import pallas as pl
from jax.experimental.pallas import tpu as pltpu
from jax.experimental.pallas import tpu_sc as plsc

_W = 4
_NUM_TILES = 32
_CHUNK = 16


def _copy_body(x_hbm, out_hbm, buf0, buf1, isem0, isem1, osem0, osem1):
    wid = lax.axis_index("s") * 2 + lax.axis_index("c")
    rows = out_hbm.shape[0] // _NUM_TILES
    base = wid * rows
    n = rows // _CHUNK
    bufs = (buf0, buf1)
    isems = (isem0, isem1)
    osems = (osem0, osem1)

    def cp_in(i):
        return pltpu.make_async_copy(
            x_hbm.at[pl.ds(base + i * _CHUNK, _CHUNK), pl.ds(0, 1)],
            bufs[i % 2], isems[i % 2])

    def cp_out(i):
        return pltpu.make_async_copy(
            bufs[i % 2], out_hbm.at[pl.ds(base + i * _CHUNK, _CHUNK)],
            osems[i % 2])

    cp_in(0).start()
    for i in range(n):
        if i + 1 < n:
            if i >= 1:
                cp_out(i - 1).wait()
            cp_in(i + 1).start()
        cp_in(i).wait()
        cp_out(i).start()
    cp_out(n - 2).wait()
    cp_out(n - 1).wait()


def kernel(x):
    b, s, d = x.shape
    h = s // _W
    xv = x.reshape(b * h, _W, d)
    mesh = plsc.VectorSubcoreMesh(core_axis_name="c", subcore_axis_name="s")
    out = pl.kernel(
        _copy_body,
        out_type=jax.ShapeDtypeStruct((b * h, 1, d), x.dtype),
        mesh=mesh,
        scratch_types=[
            pltpu.VMEM((_CHUNK, 1, d), x.dtype),
            pltpu.VMEM((_CHUNK, 1, d), x.dtype),
            pltpu.SemaphoreType.DMA,
            pltpu.SemaphoreType.DMA,
            pltpu.SemaphoreType.DMA,
            pltpu.SemaphoreType.DMA,
        ],
    )(xv)
    return out.reshape(b, h, d)


# speedup vs baseline: 6.5932x; 1.1242x over previous
import jax
import jax.numpy as jnp
from jax.experimental import pallas as pl

_W = 4
_B = 128


def _body(x_ref, o_ref):
    o_ref[...] = x_ref[...]


def kernel(x):
    b, s, d = x.shape
    h = s // _W
    n = b * h
    xv = x.reshape(n, _W * d)
    out = pl.pallas_call(
        _body,
        grid=(n // _B,),
        in_specs=[pl.BlockSpec((_B, d), lambda i: (i, 0))],
        out_specs=pl.BlockSpec((_B, d), lambda i: (i, 0)),
        out_shape=jax.ShapeDtypeStruct((n, d), x.dtype),
    )(xv)
    return out.reshape(b, h, d)
